# Initial kernel scaffold; baseline (speedup 1.0000x reference)
import jax
import jax.numpy as jnp
from jax.experimental import pallas as pl


def kernel(triangle_vertices, transmitters, receivers, mask, W_o1, b_o1, W_o2, b_o2, W_s1, b_s1, W_s2, b_s2, W_st, b_st, W_f1, b_f1, W_f2, b_f2, W_f3, b_f3):
    raise NotImplementedError("write your pallas kernel here")


# speedup vs baseline: 1.4348x; 1.4348x over previous
import jax
import jax.numpy as jnp
from jax.experimental import pallas as pl
from jax.experimental.pallas import tpu as pltpu

N = 4096
ORDER = 3
D_OBJ = 256
D_SCENE = 256
D_STATE = 128
WID = 512
EPS = 0.5


def _softplus(x):
    return jnp.maximum(x, 0.0) + jnp.log1p(jnp.exp(-jnp.abs(x)))


def _body(bern_ref, txs_ref, rxs_ref, bf3_ref,
          tv9T_ref, tx9c_ref, maskr_ref, idxr_ref,
          g0_ref, g1_ref, g2_ref,
          Wo1T_ref, bo1c_ref, Wo2T_ref, bo2c_ref,
          Ws1T_ref, bs1c_ref, Ws2T_ref, bs2c_ref,
          WstT0_ref, WstT1_ref, WstT2_ref, bstc_ref,
          Wf1aT_ref, Wf1sT_ref, Wf1stT_ref, bf1c_ref,
          Wf2T_ref, bf2c_ref, wf3r_ref,
          path_ref, loss_ref, rew_ref):
    txx, txy, txz = txs_ref[0], txs_ref[1], txs_ref[2]
    rxx, rxy, rxz = rxs_ref[0], rxs_ref[1], rxs_ref[2]
    dx, dy, dz = rxx - txx, rxy - txy, rxz - txz
    scale = jnp.sqrt(dx * dx + dy * dy + dz * dz) + 1e-6

    tv9T = tv9T_ref[...]
    xfT = (tv9T - tx9c_ref[...]) / scale
    maskr = maskr_ref[...]
    maskb = maskr > 0.5
    idxr = idxr_ref[...]

    t1 = jnp.maximum(Wo1T_ref[...] @ xfT + bo1c_ref[...], 0.0)
    objT = Wo2T_ref[...] @ t1 + bo2c_ref[...]
    objT = objT * maskr

    msum = jnp.sum(maskr)
    pooled = jnp.sum(objT, axis=1, keepdims=True) / jnp.maximum(msum, 1.0)
    sh = jnp.maximum(Ws1T_ref[...] @ pooled + bs1c_ref[...], 0.0)
    scene = Ws2T_ref[...] @ sh + bs2c_ref[...]

    h1_baseT = Wf1aT_ref[...] @ objT + (Wf1sT_ref[...] @ scene + bf1c_ref[...])

    Wf2T = Wf2T_ref[...]
    bf2c = bf2c_ref[...]
    wf3r = wf3r_ref[...]
    bf3 = bf3_ref[0]

    def flows(hpreT):
        h = jnp.maximum(hpreT, 0.0)
        h2 = jnp.maximum(Wf2T @ h + bf2c, 0.0)
        fc = wf3r @ h2 + bf3
        return jnp.where(maskb, _softplus(fc), 0.0)

    edge = flows(h1_baseT)

    gums = (g0_ref, g1_ref, g2_ref)
    WstTs = (WstT0_ref, WstT1_ref, WstT2_ref)
    loss = jnp.float32(0.0)
    prev = jnp.int32(-1)
    rowmasks = []
    for i in range(ORDER):
        unif = jnp.where(idxr == prev, 0.0, maskr)
        esum = jnp.sum(edge)
        choose_u = (bern_ref[i] != 0) | (esum == 0.0)
        policy = jnp.where(choose_u, unif, edge)
        probs = policy / jnp.maximum(jnp.sum(policy), 1e-20)
        score = jnp.log(probs + 1e-20) + gums[i][...]
        m = jnp.max(score)
        nxt = jnp.min(jnp.where(score == m, idxr, N)).astype(jnp.int32)
        path_ref[i] = nxt
        rowmask = idxr == nxt
        rowmasks.append(rowmask)
        parent = jnp.sum(jnp.where(rowmask, edge, 0.0))
        if i < ORDER - 1:
            st = bstc_ref[...]
            for j, rm in enumerate(rowmasks):
                sel = jnp.where(rm, 1.0, 0.0)
                rowv = jnp.sum(objT * sel, axis=1, keepdims=True)
                st = st + WstTs[j][...] @ rowv
            state = jnp.tanh(st)
            svc = Wf1stT_ref[...] @ state
            newe = flows(h1_baseT + svc)
            newe = jnp.where(rowmask, 0.0, newe)
            loss = loss + (parent - jnp.sum(newe)) ** 2
            edge = newe
            prev = nxt
        else:
            pts = []
            for rm in rowmasks:
                sel = jnp.where(rm, 1.0, 0.0)
                px = jnp.sum((tv9T[0:1] + tv9T[3:4] + tv9T[6:7]) * sel) / 3.0
                py = jnp.sum((tv9T[1:2] + tv9T[4:5] + tv9T[7:8]) * sel) / 3.0
                pz = jnp.sum((tv9T[2:3] + tv9T[5:6] + tv9T[8:9]) * sel) / 3.0
                pts.append((px, py, pz))
            seq = [(txx, txy, txz)] + pts + [(rxx, rxy, rxz)]
            length = jnp.float32(0.0)
            for a, b in zip(seq[:-1], seq[1:]):
                ex = b[0] - a[0] + 1e-8
                ey = b[1] - a[1] + 1e-8
                ez = b[2] - a[2] + 1e-8
                length = length + jnp.sqrt(ex * ex + ey * ey + ez * ez)
            reward = jnp.exp(-0.1 * length)
            loss = loss + (parent - reward) ** 2
            rew_ref[0] = reward
    loss_ref[0] = loss


def kernel(triangle_vertices, transmitters, receivers, mask,
           W_o1, b_o1, W_o2, b_o2, W_s1, b_s1, W_s2, b_s2,
           W_st, b_st, W_f1, b_f1, W_f2, b_f2, W_f3, b_f3):
    tv9T = triangle_vertices.reshape(N, 9).T
    txs = transmitters.reshape(3)
    rxs = receivers.reshape(3)
    tx9c = jnp.tile(txs, 3).reshape(9, 1)
    maskr = mask.astype(jnp.float32).reshape(1, N)
    idxr = jnp.arange(N, dtype=jnp.int32).reshape(1, N)

    key = jax.random.key(42)
    berns, gums = [], []
    for _ in range(ORDER):
        key, k1, k2 = jax.random.split(key, 3)
        berns.append(jax.random.bernoulli(k1, EPS))
        u = jax.random.uniform(k2, (1, N), minval=1e-9, maxval=1.0)
        gums.append(-jnp.log(-jnp.log(u)))
    bern = jnp.stack(berns).astype(jnp.int32)

    col = lambda v: v.reshape(-1, 1)
    smem = pl.BlockSpec(memory_space=pltpu.SMEM)
    vmem = pl.BlockSpec(memory_space=pltpu.VMEM)
    path, loss, rew = pl.pallas_call(
        _body,
        out_shape=(
            jax.ShapeDtypeStruct((ORDER,), jnp.int32),
            jax.ShapeDtypeStruct((1,), jnp.float32),
            jax.ShapeDtypeStruct((1,), jnp.float32),
        ),
        in_specs=[smem, smem, smem, smem] + [vmem] * 26,
        out_specs=(smem, smem, smem),
    )(bern, txs, rxs, b_f3,
      tv9T, tx9c, maskr, idxr,
      gums[0], gums[1], gums[2],
      W_o1.T, col(b_o1), W_o2.T, col(b_o2),
      W_s1.T, col(b_s1), W_s2.T, col(b_s2),
      W_st[0:D_OBJ].T, W_st[D_OBJ:2 * D_OBJ].T, W_st[2 * D_OBJ:].T, col(b_st),
      W_f1[0:D_OBJ].T, W_f1[D_OBJ:D_OBJ + D_SCENE].T,
      W_f1[D_OBJ + D_SCENE:].T, col(b_f1),
      W_f2.T, col(b_f2), W_f3.reshape(1, WID))
    return path, loss[0], rew[0]


# speedup vs baseline: 3.5704x; 2.4884x over previous
import numpy as np

import jax
import jax.numpy as jnp
from jax import lax
from jax.experimental import pallas as pl
from jax.experimental.pallas import tpu as pltpu

N = 4096
ORDER = 3
D_OBJ = 256
D_SCENE = 256
D_STATE = 128
WID = 512
EPS = 0.5

_TN = (((0,), (0,)), ((), ()))


def _tdot(a, b):
    return lax.dot_general(a, b, _TN)


def _rng_constants():
    key = jax.random.key(42)
    berns, gums = [], []
    for _ in range(ORDER):
        key, k1, k2 = jax.random.split(key, 3)
        berns.append(np.asarray(jax.random.bernoulli(k1, EPS)))
        u = jax.random.uniform(k2, (1, N), minval=1e-9, maxval=1.0)
        gums.append(np.asarray(-jnp.log(-jnp.log(u)), dtype=np.float32))
    return (np.asarray(berns, dtype=np.int32),
            np.stack(gums).reshape(ORDER, 1, N))


_BERN, _GUMS = _rng_constants()


def _softplus(x):
    return jnp.maximum(x, 0.0) + jnp.log1p(jnp.exp(-jnp.abs(x)))


def _body(bern_ref, txs_ref, rxs_ref, bf3_ref,
          tv9T_ref, tx9c_ref, maskr_ref, idxr_ref,
          g0_ref, g1_ref, g2_ref,
          Wo1_ref, bo1c_ref, Wo2_ref, bo2c_ref,
          Ws1_ref, bs1c_ref, Ws2_ref, bs2c_ref,
          Wst_ref, bstc_ref, Wf1_ref, bf1c_ref,
          Wf2_ref, bf2c_ref, Wf3_ref,
          path_ref, loss_ref, rew_ref):
    txx, txy, txz = txs_ref[0], txs_ref[1], txs_ref[2]
    rxx, rxy, rxz = rxs_ref[0], rxs_ref[1], rxs_ref[2]
    dx, dy, dz = rxx - txx, rxy - txy, rxz - txz
    scale = jnp.sqrt(dx * dx + dy * dy + dz * dz) + 1e-6

    tv9T = tv9T_ref[...]
    xfT = (tv9T - tx9c_ref[...]) / scale
    maskr = maskr_ref[...]
    maskb = maskr > 0.5
    idxr = idxr_ref[...]

    t1 = jnp.maximum(_tdot(Wo1_ref[...], xfT) + bo1c_ref[...], 0.0)
    objT = _tdot(Wo2_ref[...], t1) + bo2c_ref[...]
    objT = objT * maskr

    msum = jnp.sum(maskr)
    pooled = jnp.sum(objT, axis=1, keepdims=True) / jnp.maximum(msum, 1.0)
    sh = jnp.maximum(_tdot(Ws1_ref[...], pooled) + bs1c_ref[...], 0.0)
    scene = _tdot(Ws2_ref[...], sh) + bs2c_ref[...]

    Wf1 = Wf1_ref[...]
    h1_baseT = _tdot(Wf1[0:D_OBJ], objT) \
        + (_tdot(Wf1[D_OBJ:D_OBJ + D_SCENE], scene) + bf1c_ref[...])

    Wf2 = Wf2_ref[...]
    bf2c = bf2c_ref[...]
    Wf3 = Wf3_ref[...]
    bf3 = bf3_ref[0]

    def flows(hpreT):
        h = jnp.maximum(hpreT, 0.0)
        h2 = jnp.maximum(_tdot(Wf2, h) + bf2c, 0.0)
        fc = _tdot(Wf3, h2) + bf3
        return jnp.where(maskb, _softplus(fc), 0.0)

    edge = flows(h1_baseT)

    gums = (g0_ref, g1_ref, g2_ref)
    Wst = Wst_ref[...]
    loss = jnp.float32(0.0)
    prev = jnp.int32(-1)
    rowmasks = []
    for i in range(ORDER):
        unif = jnp.where(idxr == prev, 0.0, maskr)
        esum = jnp.sum(edge)
        choose_u = (bern_ref[i] != 0) | (esum == 0.0)
        policy = jnp.where(choose_u, unif, edge)
        probs = policy / jnp.maximum(jnp.sum(policy), 1e-20)
        score = jnp.log(probs + 1e-20) + gums[i][...]
        m = jnp.max(score)
        nxt = jnp.min(jnp.where(score == m, idxr, N)).astype(jnp.int32)
        path_ref[i] = nxt
        rowmask = idxr == nxt
        rowmasks.append(rowmask)
        parent = jnp.sum(jnp.where(rowmask, edge, 0.0))
        if i < ORDER - 1:
            st = bstc_ref[...]
            for j, rm in enumerate(rowmasks):
                sel = jnp.where(rm, 1.0, 0.0)
                rowv = jnp.sum(objT * sel, axis=1, keepdims=True)
                st = st + _tdot(Wst[D_OBJ * j:D_OBJ * (j + 1)], rowv)
            state = jnp.tanh(st)
            svc = _tdot(Wf1[D_OBJ + D_SCENE:], state)
            newe = flows(h1_baseT + svc)
            newe = jnp.where(rowmask, 0.0, newe)
            loss = loss + (parent - jnp.sum(newe)) ** 2
            edge = newe
            prev = nxt
        else:
            pts = []
            for rm in rowmasks:
                sel = jnp.where(rm, 1.0, 0.0)
                px = jnp.sum((tv9T[0:1] + tv9T[3:4] + tv9T[6:7]) * sel) / 3.0
                py = jnp.sum((tv9T[1:2] + tv9T[4:5] + tv9T[7:8]) * sel) / 3.0
                pz = jnp.sum((tv9T[2:3] + tv9T[5:6] + tv9T[8:9]) * sel) / 3.0
                pts.append((px, py, pz))
            seq = [(txx, txy, txz)] + pts + [(rxx, rxy, rxz)]
            length = jnp.float32(0.0)
            for a, b in zip(seq[:-1], seq[1:]):
                ex = b[0] - a[0] + 1e-8
                ey = b[1] - a[1] + 1e-8
                ez = b[2] - a[2] + 1e-8
                length = length + jnp.sqrt(ex * ex + ey * ey + ez * ez)
            reward = jnp.exp(-0.1 * length)
            loss = loss + (parent - reward) ** 2
            rew_ref[0] = reward
    loss_ref[0] = loss


def kernel(triangle_vertices, transmitters, receivers, mask,
           W_o1, b_o1, W_o2, b_o2, W_s1, b_s1, W_s2, b_s2,
           W_st, b_st, W_f1, b_f1, W_f2, b_f2, W_f3, b_f3):
    tv9T = triangle_vertices.reshape(N, 9).T
    txs = transmitters.reshape(3)
    rxs = receivers.reshape(3)
    tx9c = jnp.tile(txs, 3).reshape(9, 1)
    maskr = mask.astype(jnp.float32).reshape(1, N)
    idxr = np.arange(N, dtype=np.int32).reshape(1, N)

    col = lambda v: v.reshape(-1, 1)
    smem = pl.BlockSpec(memory_space=pltpu.SMEM)
    vmem = pl.BlockSpec(memory_space=pltpu.VMEM)
    path, loss, rew = pl.pallas_call(
        _body,
        out_shape=(
            jax.ShapeDtypeStruct((ORDER,), jnp.int32),
            jax.ShapeDtypeStruct((1,), jnp.float32),
            jax.ShapeDtypeStruct((1,), jnp.float32),
        ),
        in_specs=[smem, smem, smem, smem] + [vmem] * 22,
        out_specs=(smem, smem, smem),
    )(_BERN, txs, rxs, b_f3,
      tv9T, tx9c, maskr, idxr,
      _GUMS[0], _GUMS[1], _GUMS[2],
      W_o1, col(b_o1), W_o2, col(b_o2),
      W_s1, col(b_s1), W_s2, col(b_s2),
      W_st, col(b_st), W_f1, col(b_f1),
      W_f2, col(b_f2), W_f3)
    return path, loss[0], rew[0]
